# Initial kernel scaffold; baseline (speedup 1.0000x reference)
import jax
import jax.numpy as jnp
from jax.experimental import pallas as pl


def kernel(x, y, We, be, Wg, bg):
    raise NotImplementedError("write your pallas kernel here")


# speedup vs baseline: 1.5113x; 1.5113x over previous
import functools

import jax
import jax.numpy as jnp
from jax import lax
from jax.experimental import pallas as pl
from jax.experimental.pallas import tpu as pltpu
from jax.experimental.pallas import tpu_sc as plsc

N = 4096
D = 2048
E = 8
K = 2
EP = 128
M = N * K
TILE = 256
NB = M // TILE + E
MPAD = NB * TILE

NW = 32
GROWS = MPAD // NW
GCH = 32
CTOK = N // NW
CCH = 16


def _gating_body(x_ref, wg_ref, bg_ref, idx_ref, gate_ref):
    x = x_ref[...]
    wg = wg_ref[...]
    logits = lax.dot_general(
        x, wg, (((1,), (1,)), ((), ())),
        preferred_element_type=jnp.float32,
        precision=lax.Precision.DEFAULT,
    ) + bg_ref[...]
    lane = lax.broadcasted_iota(jnp.int32, logits.shape, 1)
    v0 = jnp.max(logits, axis=1, keepdims=True)
    i0 = jnp.min(jnp.where(logits == v0, lane, EP), axis=1, keepdims=True)
    l2 = jnp.where(lane == i0, jnp.float32(-1e30), logits)
    v1 = jnp.max(l2, axis=1, keepdims=True)
    i1 = jnp.min(jnp.where(l2 == v1, lane, EP), axis=1, keepdims=True)
    t = jnp.exp(v1 - v0)
    g0 = 1.0 / (1.0 + t)
    g1 = t / (1.0 + t)
    idx_ref[...] = jnp.where(lane == 0, i0, jnp.where(lane == 1, i1, 0))
    gate_ref[...] = jnp.where(lane == 0, g0, jnp.where(lane == 1, g1, 0.0))


def _gating(inp, Wg, bg):
    wgp = jnp.zeros((EP, D), jnp.float32).at[:E].set(Wg)
    bgp = jnp.full((1, EP), -1e30, jnp.float32).at[0, :E].set(bg)
    bn = 1024
    idx_out, gate_out = pl.pallas_call(
        _gating_body,
        grid=(N // bn,),
        in_specs=[
            pl.BlockSpec((bn, D), lambda b: (b, 0)),
            pl.BlockSpec((EP, D), lambda b: (0, 0)),
            pl.BlockSpec((1, EP), lambda b: (0, 0)),
        ],
        out_specs=[
            pl.BlockSpec((bn, EP), lambda b: (b, 0)),
            pl.BlockSpec((bn, EP), lambda b: (b, 0)),
        ],
        out_shape=[
            jax.ShapeDtypeStruct((N, EP), jnp.int32),
            jax.ShapeDtypeStruct((N, EP), jnp.float32),
        ],
    )(inp, wgp, bgp)
    return idx_out[:, :K], gate_out[:, :K]


def _gather_body(src_hbm, gidx_hbm, out_hbm, idx_v, rows_v, sem):
    wid = lax.axis_index("s") * 2 + lax.axis_index("c")
    base = wid * GROWS
    for c in range(GROWS // GCH):
        pltpu.sync_copy(gidx_hbm.at[pl.ds(base + c * GCH, GCH)], idx_v)
        pltpu.async_copy(src_hbm.at[idx_v], rows_v, sem).wait()
        pltpu.sync_copy(rows_v, out_hbm.at[pl.ds(base + c * GCH, GCH)])


def _dispatch_gather(inp, gather_src):
    mesh = plsc.VectorSubcoreMesh(core_axis_name="c", subcore_axis_name="s")
    fn = pl.kernel(
        _gather_body,
        out_type=jax.ShapeDtypeStruct((MPAD, D), jnp.float32),
        mesh=mesh,
        scratch_types=[
            pltpu.VMEM((GCH,), jnp.int32),
            pltpu.VMEM((GCH, D), jnp.float32),
            pltpu.SemaphoreType.DMA,
        ],
    )
    return fn(inp, gather_src)


def _gemm_body(bexp_ref, x_ref, w_ref, b_ref, g_ref, y_ref):
    del bexp_ref
    x = x_ref[...]
    w = w_ref[0]
    acc = lax.dot_general(
        x, w, (((1,), (1,)), ((), ())),
        preferred_element_type=jnp.float32,
        precision=lax.Precision.DEFAULT,
    )
    y_ref[...] = (acc + b_ref[0]) * g_ref[...]


def _grouped_gemm(Xg, We, be, blk_exp, slot_gate):
    grid_spec = pltpu.PrefetchScalarGridSpec(
        num_scalar_prefetch=1,
        grid=(NB,),
        in_specs=[
            pl.BlockSpec((TILE, D), lambda b, s: (b, 0)),
            pl.BlockSpec((1, D, D), lambda b, s: (s[b], 0, 0)),
            pl.BlockSpec((1, 1, D), lambda b, s: (s[b], 0, 0)),
            pl.BlockSpec((TILE, 1), lambda b, s: (b, 0)),
        ],
        out_specs=pl.BlockSpec((TILE, D), lambda b, s: (b, 0)),
    )
    return pl.pallas_call(
        _gemm_body,
        grid_spec=grid_spec,
        out_shape=jax.ShapeDtypeStruct((MPAD, D), jnp.float32),
    )(blk_exp, Xg, We, be.reshape(E, 1, D), slot_gate.reshape(MPAD, 1))


def _combine_body(y_hbm, pos_hbm, out_hbm, idx_v, rows_v, out_v, sem):
    wid = lax.axis_index("s") * 2 + lax.axis_index("c")
    base = wid * CTOK
    for c in range(CTOK // CCH):
        pltpu.sync_copy(
            pos_hbm.at[pl.ds(K * (base + c * CCH), K * CCH)], idx_v)
        pltpu.async_copy(y_hbm.at[idx_v], rows_v, sem).wait()

        def jbody(j, carry):
            off = j * 16
            for t in range(CCH):
                a = rows_v[2 * t, pl.ds(off, 16)]
                b = rows_v[2 * t + 1, pl.ds(off, 16)]
                out_v[t, pl.ds(off, 16)] = a + b
            return carry

        lax.fori_loop(0, D // 16, jbody, 0)
        pltpu.sync_copy(out_v, out_hbm.at[pl.ds(base + c * CCH, CCH)])


def _combine(Y, pos):
    mesh = plsc.VectorSubcoreMesh(core_axis_name="c", subcore_axis_name="s")
    fn = pl.kernel(
        _combine_body,
        out_type=jax.ShapeDtypeStruct((N, D), jnp.float32),
        mesh=mesh,
        scratch_types=[
            pltpu.VMEM((K * CCH,), jnp.int32),
            pltpu.VMEM((K * CCH, D), jnp.float32),
            pltpu.VMEM((CCH, D), jnp.float32),
            pltpu.SemaphoreType.DMA,
        ],
    )
    return fn(Y, pos)


def _route(idx2, gates2):
    e_flat = idx2.reshape(M)
    g_flat = gates2.reshape(M)
    onehot = (e_flat[:, None] == jnp.arange(E, dtype=jnp.int32)[None, :])
    oh = onehot.astype(jnp.int32)
    counts = jnp.sum(oh, axis=0)
    rank = jnp.sum(jnp.where(onehot, jnp.cumsum(oh, axis=0) - oh, 0), axis=1)
    nblk = (counts + TILE - 1) // TILE
    cum = jnp.cumsum(nblk)
    blk_off = cum - nblk
    pos = blk_off[e_flat] * TILE + rank
    gather_src = jnp.zeros((MPAD,), jnp.int32).at[pos].set(
        jnp.arange(M, dtype=jnp.int32) // K)
    slot_gate = jnp.zeros((MPAD,), jnp.float32).at[pos].set(g_flat)
    bids = jnp.arange(NB, dtype=jnp.int32)
    blk_exp = jnp.minimum(
        jnp.sum((bids[:, None] >= cum[None, :]).astype(jnp.int32), axis=1),
        E - 1)
    return gather_src, slot_gate, blk_exp, pos


def kernel(x, y, We, be, Wg, bg):
    inp = jnp.concatenate([x, y], axis=1)
    idx2, gates2 = _gating(inp, Wg, bg)
    gather_src, slot_gate, blk_exp, pos = _route(idx2, gates2)
    Xg = _dispatch_gather(inp, gather_src)
    Y = _grouped_gemm(Xg, We, be, blk_exp, slot_gate)
    return _combine(Y, pos)


# speedup vs baseline: 1.5300x; 1.0123x over previous
import functools

import jax
import jax.numpy as jnp
from jax import lax
from jax.experimental import pallas as pl
from jax.experimental.pallas import tpu as pltpu
from jax.experimental.pallas import tpu_sc as plsc

N = 4096
D = 2048
E = 8
K = 2
EP = 128
M = N * K
TILE = 256
NB = M // TILE + E
MPAD = NB * TILE

NW = 32
GROWS = MPAD // NW
GCH = 16
GNCH = GROWS // GCH
CTOK = N // NW
CCH = 8
CNCH = CTOK // CCH


def _gating_body(x_ref, wg_ref, bg_ref, idx_ref, gate_ref):
    x = x_ref[...]
    wg = wg_ref[...]
    logits = lax.dot_general(
        x, wg, (((1,), (1,)), ((), ())),
        preferred_element_type=jnp.float32,
        precision=lax.Precision.DEFAULT,
    ) + bg_ref[...]
    lane = lax.broadcasted_iota(jnp.int32, logits.shape, 1)
    v0 = jnp.max(logits, axis=1, keepdims=True)
    i0 = jnp.min(jnp.where(logits == v0, lane, EP), axis=1, keepdims=True)
    l2 = jnp.where(lane == i0, jnp.float32(-1e30), logits)
    v1 = jnp.max(l2, axis=1, keepdims=True)
    i1 = jnp.min(jnp.where(l2 == v1, lane, EP), axis=1, keepdims=True)
    t = jnp.exp(v1 - v0)
    g0 = 1.0 / (1.0 + t)
    g1 = t / (1.0 + t)
    idx_ref[...] = jnp.where(lane == 0, i0, jnp.where(lane == 1, i1, 0))
    gate_ref[...] = jnp.where(lane == 0, g0, jnp.where(lane == 1, g1, 0.0))


def _gating(inp, Wg, bg):
    wgp = jnp.zeros((EP, D), jnp.float32).at[:E].set(Wg)
    bgp = jnp.full((1, EP), -1e30, jnp.float32).at[0, :E].set(bg)
    bn = 1024
    idx_out, gate_out = pl.pallas_call(
        _gating_body,
        grid=(N // bn,),
        in_specs=[
            pl.BlockSpec((bn, D), lambda b: (b, 0)),
            pl.BlockSpec((EP, D), lambda b: (0, 0)),
            pl.BlockSpec((1, EP), lambda b: (0, 0)),
        ],
        out_specs=[
            pl.BlockSpec((bn, EP), lambda b: (b, 0)),
            pl.BlockSpec((bn, EP), lambda b: (b, 0)),
        ],
        out_shape=[
            jax.ShapeDtypeStruct((N, EP), jnp.int32),
            jax.ShapeDtypeStruct((N, EP), jnp.float32),
        ],
    )(inp, wgp, bgp)
    return idx_out[:, :K], gate_out[:, :K]


def _gather_body(src_hbm, gidx_hbm, out_hbm, idx0, idx1, rows_v, sem0, sem1):
    wid = lax.axis_index("s") * 2 + lax.axis_index("c")
    base = wid * GROWS
    idxs = (idx0, idx1)
    sems = (sem0, sem1)
    pending = [None, None]
    pltpu.sync_copy(gidx_hbm.at[pl.ds(base, GCH)], idx0)
    pending[0] = pltpu.async_copy(src_hbm.at[idx0], rows_v.at[0], sem0)
    for c in range(GNCH):
        b = c % 2
        nb = (c + 1) % 2
        if c + 1 < GNCH:
            pltpu.sync_copy(
                gidx_hbm.at[pl.ds(base + (c + 1) * GCH, GCH)], idxs[nb])
            pending[nb] = pltpu.async_copy(
                src_hbm.at[idxs[nb]], rows_v.at[nb], sems[nb])
        pending[b].wait()
        pltpu.sync_copy(rows_v.at[b], out_hbm.at[pl.ds(base + c * GCH, GCH)])


def _dispatch_gather(inp, gather_src):
    mesh = plsc.VectorSubcoreMesh(core_axis_name="c", subcore_axis_name="s")
    fn = pl.kernel(
        _gather_body,
        out_type=jax.ShapeDtypeStruct((MPAD, D), jnp.float32),
        mesh=mesh,
        scratch_types=[
            pltpu.VMEM((GCH,), jnp.int32),
            pltpu.VMEM((GCH,), jnp.int32),
            pltpu.VMEM((2, GCH, D), jnp.float32),
            pltpu.SemaphoreType.DMA,
            pltpu.SemaphoreType.DMA,
        ],
    )
    return fn(inp, gather_src)


def _gemm_body(bw_ref, act_ref, x_ref, w_ref, b_ref, g_ref, y_ref):
    del bw_ref
    blk = pl.program_id(0)

    @pl.when(act_ref[blk] > 0)
    def _():
        x = x_ref[...]
        w = w_ref[0]
        acc = lax.dot_general(
            x, w, (((1,), (1,)), ((), ())),
            preferred_element_type=jnp.float32,
            precision=lax.Precision.DEFAULT,
        )
        y_ref[...] = (acc + b_ref[0]) * g_ref[...]


def _grouped_gemm(Xg, We, be, blk_weight, blk_active, slot_gate):
    grid_spec = pltpu.PrefetchScalarGridSpec(
        num_scalar_prefetch=2,
        grid=(NB,),
        in_specs=[
            pl.BlockSpec((TILE, D), lambda b, s, a: (b, 0)),
            pl.BlockSpec((1, D, D), lambda b, s, a: (s[b], 0, 0)),
            pl.BlockSpec((1, 1, D), lambda b, s, a: (s[b], 0, 0)),
            pl.BlockSpec((TILE, 1), lambda b, s, a: (b, 0)),
        ],
        out_specs=pl.BlockSpec((TILE, D), lambda b, s, a: (b, 0)),
    )
    return pl.pallas_call(
        _gemm_body,
        grid_spec=grid_spec,
        out_shape=jax.ShapeDtypeStruct((MPAD, D), jnp.float32),
    )(blk_weight, blk_active, Xg, We, be.reshape(E, 1, D),
      slot_gate.reshape(MPAD, 1))


def _combine_body(y_hbm, pos_hbm, out_hbm, idx0, idx1, rows_v, out_v,
                  sem0, sem1):
    wid = lax.axis_index("s") * 2 + lax.axis_index("c")
    base = wid * CTOK
    idxs = (idx0, idx1)
    sems = (sem0, sem1)
    pending = [None, None]
    pltpu.sync_copy(pos_hbm.at[pl.ds(K * base, K * CCH)], idx0)
    pending[0] = pltpu.async_copy(y_hbm.at[idx0], rows_v.at[0], sem0)
    for c in range(CNCH):
        b = c % 2
        nb = (c + 1) % 2
        if c + 1 < CNCH:
            pltpu.sync_copy(
                pos_hbm.at[pl.ds(K * (base + (c + 1) * CCH), K * CCH)],
                idxs[nb])
            pending[nb] = pltpu.async_copy(
                y_hbm.at[idxs[nb]], rows_v.at[nb], sems[nb])
        pending[b].wait()

        def jbody(j, carry):
            off = j * 16
            for t in range(CCH):
                a = rows_v[b, 2 * t, pl.ds(off, 16)]
                bb = rows_v[b, 2 * t + 1, pl.ds(off, 16)]
                out_v[b, t, pl.ds(off, 16)] = a + bb
            return carry

        lax.fori_loop(0, D // 16, jbody, 0)
        pltpu.sync_copy(out_v.at[b], out_hbm.at[pl.ds(base + c * CCH, CCH)])


def _combine(Y, pos):
    mesh = plsc.VectorSubcoreMesh(core_axis_name="c", subcore_axis_name="s")
    fn = pl.kernel(
        _combine_body,
        out_type=jax.ShapeDtypeStruct((N, D), jnp.float32),
        mesh=mesh,
        scratch_types=[
            pltpu.VMEM((K * CCH,), jnp.int32),
            pltpu.VMEM((K * CCH,), jnp.int32),
            pltpu.VMEM((2, K * CCH, D), jnp.float32),
            pltpu.VMEM((2, CCH, D), jnp.float32),
            pltpu.SemaphoreType.DMA,
            pltpu.SemaphoreType.DMA,
        ],
    )
    return fn(Y, pos)


def _route(idx2, gates2):
    e_flat = idx2.reshape(M)
    g_flat = gates2.reshape(M)
    onehot = (e_flat[:, None] == jnp.arange(E, dtype=jnp.int32)[None, :])
    oh = onehot.astype(jnp.int32)
    counts = jnp.sum(oh, axis=0)
    rank = jnp.sum(jnp.where(onehot, jnp.cumsum(oh, axis=0) - oh, 0), axis=1)
    nblk = (counts + TILE - 1) // TILE
    cum = jnp.cumsum(nblk)
    blk_off = cum - nblk
    pos = blk_off[e_flat] * TILE + rank
    gather_src = jnp.zeros((MPAD,), jnp.int32).at[pos].set(
        jnp.arange(M, dtype=jnp.int32) // K)
    slot_gate = jnp.zeros((MPAD,), jnp.float32).at[pos].set(g_flat)
    bids = jnp.arange(NB, dtype=jnp.int32)
    blk_exp = jnp.minimum(
        jnp.sum((bids[:, None] >= cum[None, :]).astype(jnp.int32), axis=1),
        E - 1)
    blk_active = (bids < cum[E - 1]).astype(jnp.int32)
    last_exp = jnp.max(jnp.where(counts > 0,
                                 jnp.arange(E, dtype=jnp.int32), 0))
    blk_weight = jnp.where(blk_active > 0, blk_exp, last_exp)
    return gather_src, slot_gate, blk_weight, blk_active, pos


def kernel(x, y, We, be, Wg, bg):
    inp = jnp.concatenate([x, y], axis=1)
    idx2, gates2 = _gating(inp, Wg, bg)
    gather_src, slot_gate, blk_weight, blk_active, pos = _route(idx2, gates2)
    Xg = _dispatch_gather(inp, gather_src)
    Y = _grouped_gemm(Xg, We, be, blk_weight, blk_active, slot_gate)
    return _combine(Y, pos)
